# baseline (speedup 1.0000x reference)
import functools

import jax
import jax.numpy as jnp
from jax import lax
from jax.experimental import pallas as pl
from jax.experimental.pallas import tpu as pltpu
from jax.experimental.pallas import tpu_sc as plsc

N = 10000
E = 320000
IN_SIZE = 128
HID = 16
OUT_SIZE = 64
HEADS = 8

NC = 2
NS = 16
NW = NC * NS
EPT = E // NW
RB = 624
TAIL0 = NS * RB
TAILN = N - TAIL0
KA = 1000
KB = 80

_f32 = jnp.float32
_i32 = jnp.int32


def _vgather(x, idx):
    return lax.gather(
        x, idx[:, None],
        dimension_numbers=lax.GatherDimensionNumbers(
            offset_dims=(), collapsed_slice_dims=(0,), start_index_map=(0,)),
        slice_sizes=(1,),
        mode=lax.GatherScatterMode.PROMISE_IN_BOUNDS)


def _dense1(x, W1, al1, ar1, S8):
    R = 1000
    grid = N // R

    def body(x_r, w_r, al_r, ar_r, s8_r, h_r, tab_r, gm_r):
        pid = pl.program_id(0)
        h = jnp.dot(x_r[...], w_r[...], preferred_element_type=_f32)
        h_r[...] = h
        el = jnp.dot(h * al_r[...], s8_r[...], preferred_element_type=_f32)
        er = jnp.dot(h * ar_r[...], s8_r[...], preferred_element_type=_f32)
        tab_r[...] = jnp.concatenate([el, er], axis=1)
        m = jnp.max(el, axis=0, keepdims=True)
        rowb = jnp.broadcast_to(jnp.concatenate([m, m], axis=1), (8, 16))

        @pl.when(pid == 0)
        def _():
            gm_r[...] = rowb

        @pl.when(pid != 0)
        def _():
            gm_r[...] = jnp.maximum(gm_r[...], rowb)

    return pl.pallas_call(
        body,
        grid=(grid,),
        in_specs=[
            pl.BlockSpec((R, IN_SIZE), lambda i: (i, 0)),
            pl.BlockSpec((IN_SIZE, IN_SIZE), lambda i: (0, 0)),
            pl.BlockSpec((1, IN_SIZE), lambda i: (0, 0)),
            pl.BlockSpec((1, IN_SIZE), lambda i: (0, 0)),
            pl.BlockSpec((IN_SIZE, 8), lambda i: (0, 0)),
        ],
        out_specs=[
            pl.BlockSpec((R, IN_SIZE), lambda i: (i, 0)),
            pl.BlockSpec((R, 16), lambda i: (i, 0)),
            pl.BlockSpec((8, 16), lambda i: (0, 0)),
        ],
        out_shape=[
            jax.ShapeDtypeStruct((N, IN_SIZE), _f32),
            jax.ShapeDtypeStruct((N, 16), _f32),
            jax.ShapeDtypeStruct((8, 16), _f32),
        ],
    )(x, W1, al1, ar1, S8)


def _pass_a(src, dst, tab, gmax, z16):
    mesh = plsc.VectorSubcoreMesh(core_axis_name="c", subcore_axis_name="s")

    @functools.partial(
        pl.kernel,
        out_type=[jax.ShapeDtypeStruct((E, 16), _f32),
                  jax.ShapeDtypeStruct((NC, N, 16), _f32)],
        mesh=mesh,
        compiler_params=pltpu.CompilerParams(use_tc_tiling_on_sc=False),
        scratch_types=[
            pltpu.VMEM((KA,), _i32), pltpu.VMEM((KA,), _i32),
            pltpu.VMEM((KA, 16), _f32), pltpu.VMEM((KA, 16), _f32),
            pltpu.VMEM((KA, 16), _f32), pltpu.VMEM((16,), _f32),
            pltpu.VMEM_SHARED((N, 16), _f32),
            pltpu.SemaphoreType.DMA, pltpu.SemaphoreType.DMA,
        ],
    )
    def k(src_h, dst_h, tab_h, gm_h, z_h, ehat_h, s_h,
          srcb, dstb, srows, drows, ebuf, gbuf, sacc, sem1, sem2):
        cid = lax.axis_index("c")
        sid = lax.axis_index("s")
        w = sid * NC + cid
        r0 = sid * RB
        pltpu.sync_copy(z_h.at[pl.ds(r0, RB)], sacc.at[pl.ds(r0, RB)])

        @pl.when(sid == 0)
        def _():
            pltpu.sync_copy(z_h.at[pl.ds(TAIL0, TAILN)],
                            sacc.at[pl.ds(TAIL0, TAILN)])

        pltpu.sync_copy(gm_h.at[0], gbuf)
        plsc.subcore_barrier()

        lanes = lax.broadcasted_iota(_i32, (16,), 0)
        sel = lanes < 8
        i07 = lax.bitwise_and(lanes, 7)
        i7p8 = i07 + 8
        gv = gbuf[...]

        def chunk(j, carry):
            off = w * EPT + j * KA
            pltpu.sync_copy(src_h.at[pl.ds(off, KA)], srcb)
            pltpu.sync_copy(dst_h.at[pl.ds(off, KA)], dstb)
            cp1 = pltpu.async_copy(tab_h.at[srcb], srows, sem1)
            cp2 = pltpu.async_copy(tab_h.at[dstb], drows, sem2)
            cp1.wait()
            cp2.wait()

            def pair(i, c2):
                s0 = srows[2 * i]
                s1 = srows[2 * i + 1]
                d0 = drows[2 * i]
                d1 = drows[2 * i + 1]
                el2 = jnp.where(sel, s0, _vgather(s1, i07))
                er2 = jnp.where(sel, _vgather(d0, i7p8), d1)
                z = el2 + er2
                e = jnp.where(z >= 0.0, z, 0.2 * z)
                mh = jnp.maximum(gv + er2, 0.0)
                eh = jnp.exp(e - mh)
                ebuf[2 * i] = jnp.where(sel, eh, _vgather(eh, i07))
                ebuf[2 * i + 1] = jnp.where(sel, _vgather(eh, i7p8), eh)
                return c2

            lax.fori_loop(0, KA // 2, pair, 0)
            pltpu.sync_copy(ebuf, ehat_h.at[pl.ds(off, KA)])
            pltpu.sync_copy(ebuf, sacc.at[dstb], add=True)
            return carry

        lax.fori_loop(0, EPT // KA, chunk, 0)
        plsc.subcore_barrier()
        pltpu.sync_copy(sacc.at[pl.ds(r0, RB)],
                        s_h.at[cid, pl.ds(r0, RB)])

        @pl.when(sid == 0)
        def _():
            pltpu.sync_copy(sacc.at[pl.ds(TAIL0, TAILN)],
                            s_h.at[cid, pl.ds(TAIL0, TAILN)])

    return k(src, dst, tab, gmax, z16)


def _recip(s):
    R = 1000
    grid = N // R

    def body(a_r, b_r, o_r):
        o_r[...] = 1.0 / (a_r[...] + b_r[...] + 1e-9)

    return pl.pallas_call(
        body,
        grid=(grid,),
        in_specs=[pl.BlockSpec((R, 16), lambda i: (i, 0)),
                  pl.BlockSpec((R, 16), lambda i: (i, 0))],
        out_specs=pl.BlockSpec((R, 16), lambda i: (i, 0)),
        out_shape=jax.ShapeDtypeStruct((N, 16), _f32),
    )(s[0], s[1])


def _pass_b(src, dst, ehat, rs, tables, z128, heads_of):
    C = len(tables)
    mesh = plsc.VectorSubcoreMesh(core_axis_name="c", subcore_axis_name="s")

    @functools.partial(
        pl.kernel,
        out_type=[jax.ShapeDtypeStruct((NC, N, 128), _f32) for _ in range(C)],
        mesh=mesh,
        compiler_params=pltpu.CompilerParams(use_tc_tiling_on_sc=False),
        scratch_types=[
            pltpu.VMEM((KB,), _i32), pltpu.VMEM((KB,), _i32),
            pltpu.VMEM((KB, 128), _f32), pltpu.VMEM((KB, 128), _f32),
            pltpu.VMEM((KB, 16), _f32), pltpu.VMEM((KB, 16), _f32),
            pltpu.VMEM_SHARED((N, 128), _f32),
            pltpu.SemaphoreType.DMA, pltpu.SemaphoreType.DMA,
        ],
    )
    def k(src_h, dst_h, ehat_h, rs_h, *rest):
        tabs = rest[:C]
        z_h = rest[C]
        outs = rest[C + 1:2 * C + 1]
        (srcb, dstb, hrows, msgb, ebuf, rsrows, acc, sem1, sem2) = rest[2 * C + 1:]
        cid = lax.axis_index("c")
        sid = lax.axis_index("s")
        w = sid * NC + cid
        r0 = sid * RB
        splats = [jnp.full((16,), h, _i32) for h in range(HEADS)]

        for c in range(C):
            pltpu.sync_copy(z_h.at[pl.ds(r0, RB)], acc.at[pl.ds(r0, RB)])

            @pl.when(sid == 0)
            def _():
                pltpu.sync_copy(z_h.at[pl.ds(TAIL0, TAILN)],
                                acc.at[pl.ds(TAIL0, TAILN)])

            plsc.subcore_barrier()

            def echunk(j, carry, _c=c):
                off = w * EPT + j * KB
                pltpu.sync_copy(src_h.at[pl.ds(off, KB)], srcb)
                pltpu.sync_copy(dst_h.at[pl.ds(off, KB)], dstb)
                cph = pltpu.async_copy(tabs[_c].at[srcb], hrows, sem1)
                cpr = pltpu.async_copy(rs_h.at[dstb], rsrows, sem2)
                pltpu.sync_copy(ehat_h.at[pl.ds(off, KB)], ebuf)
                cph.wait()
                cpr.wait()

                def edge(kk, c2):
                    arow = ebuf[kk] * rsrows[kk]
                    for jj in range(8):
                        sp = _vgather(arow, splats[heads_of[_c][jj]])
                        msgb[kk, pl.ds(16 * jj, 16)] = (
                            hrows[kk, pl.ds(16 * jj, 16)] * sp)
                    return c2

                lax.fori_loop(0, KB, edge, 0)
                pltpu.sync_copy(msgb, acc.at[dstb], add=True)
                return carry

            lax.fori_loop(0, EPT // KB, echunk, 0)
            plsc.subcore_barrier()
            pltpu.sync_copy(acc.at[pl.ds(r0, RB)],
                            outs[c].at[cid, pl.ds(r0, RB)])

            @pl.when(sid == 0)
            def _():
                pltpu.sync_copy(acc.at[pl.ds(TAIL0, TAILN)],
                                outs[c].at[cid, pl.ds(TAIL0, TAILN)])

    return k(src, dst, ehat, rs, *tables, z128)


def _dense2(p0, p1, b1, W2, al2, ar2, S8):
    R = 1000
    grid = N // R

    def body(p0_r, p1_r, b1_r, w2_r, al_r, ar_r, s8_r,
             h0_r, h1_r, h2_r, h3_r, tab_r, gm_r):
        pid = pl.program_id(0)
        o1 = jnp.maximum(p0_r[...] + p1_r[...] + b1_r[...], 0.0)
        h2 = jnp.dot(o1, w2_r[...], preferred_element_type=_f32)
        h0_r[...] = h2[:, 0:128]
        h1_r[...] = h2[:, 128:256]
        h2_r[...] = h2[:, 256:384]
        h3_r[...] = h2[:, 384:512]
        el = jnp.dot(h2 * al_r[...], s8_r[...], preferred_element_type=_f32)
        er = jnp.dot(h2 * ar_r[...], s8_r[...], preferred_element_type=_f32)
        tab_r[...] = jnp.concatenate([el, er], axis=1)
        m = jnp.max(el, axis=0, keepdims=True)
        rowb = jnp.broadcast_to(jnp.concatenate([m, m], axis=1), (8, 16))

        @pl.when(pid == 0)
        def _():
            gm_r[...] = rowb

        @pl.when(pid != 0)
        def _():
            gm_r[...] = jnp.maximum(gm_r[...], rowb)

    F = HEADS * OUT_SIZE
    return pl.pallas_call(
        body,
        grid=(grid,),
        in_specs=[
            pl.BlockSpec((R, 128), lambda i: (i, 0)),
            pl.BlockSpec((R, 128), lambda i: (i, 0)),
            pl.BlockSpec((1, 128), lambda i: (0, 0)),
            pl.BlockSpec((128, F), lambda i: (0, 0)),
            pl.BlockSpec((1, F), lambda i: (0, 0)),
            pl.BlockSpec((1, F), lambda i: (0, 0)),
            pl.BlockSpec((F, 8), lambda i: (0, 0)),
        ],
        out_specs=[pl.BlockSpec((R, 128), lambda i: (i, 0)) for _ in range(4)]
        + [pl.BlockSpec((R, 16), lambda i: (i, 0)),
           pl.BlockSpec((8, 16), lambda i: (0, 0))],
        out_shape=[jax.ShapeDtypeStruct((N, 128), _f32) for _ in range(4)]
        + [jax.ShapeDtypeStruct((N, 16), _f32),
           jax.ShapeDtypeStruct((8, 16), _f32)],
    )(p0, p1, b1, W2, al2, ar2, S8)


def _final(parts, b2c, M128):
    R = 1000
    grid = N // R

    def body(q00, q01, q10, q11, q20, q21, q30, q31, bc_r, m_r, o_r):
        qs = [(q00, q01), (q10, q11), (q20, q21), (q30, q31)]
        z = jnp.zeros((R, OUT_SIZE), _f32)
        for c in range(4):
            a, b = qs[c]
            t = a[...] + b[...] + bc_r[pl.ds(c, 1), :]
            z = z + jnp.dot(t, m_r[...], preferred_element_type=_f32)
        t = z - jnp.max(z, axis=1, keepdims=True)
        o_r[...] = t - jnp.log(jnp.sum(jnp.exp(t), axis=1, keepdims=True))

    return pl.pallas_call(
        body,
        grid=(grid,),
        in_specs=[pl.BlockSpec((R, 128), lambda i: (i, 0))
                  for _ in range(8)]
        + [pl.BlockSpec((4, 128), lambda i: (0, 0)),
           pl.BlockSpec((128, OUT_SIZE), lambda i: (0, 0))],
        out_specs=pl.BlockSpec((R, OUT_SIZE), lambda i: (i, 0)),
        out_shape=jax.ShapeDtypeStruct((N, OUT_SIZE), _f32),
    )(*parts, b2c, M128)


def kernel(features, edge_index, W1, al1, ar1, b1, W2, al2, ar2, b2):
    src = edge_index[0].astype(_i32)
    dst = edge_index[1].astype(_i32)

    al1r = al1.reshape(1, HEADS * HID)
    ar1r = ar1.reshape(1, HEADS * HID)
    al2r = al2.reshape(1, HEADS * OUT_SIZE)
    ar2r = ar2.reshape(1, HEADS * OUT_SIZE)
    b1r = b1.reshape(1, HEADS * HID)
    b2c = b2.reshape(4, 128)
    hid_sel = jnp.equal(
        jnp.arange(HEADS * HID)[:, None] // HID,
        jnp.arange(HEADS)[None, :]).astype(_f32)
    out_sel = jnp.equal(
        jnp.arange(HEADS * OUT_SIZE)[:, None] // OUT_SIZE,
        jnp.arange(HEADS)[None, :]).astype(_f32)
    mean_m = jnp.tile(jnp.eye(OUT_SIZE, dtype=_f32), (2, 1)) / HEADS
    z16 = jnp.zeros((N, 16), _f32)
    z128 = jnp.zeros((N, 128), _f32)

    h1, tab1, gm1 = _dense1(features, W1, al1r, ar1r, hid_sel)
    ehat1, s1 = _pass_a(src, dst, tab1, gm1, z16)
    rs1 = _recip(s1)
    (p1,) = _pass_b(src, dst, ehat1, rs1, [h1], z128,
                    heads_of=[list(range(8))])

    h2c = _dense2(p1[0], p1[1], b1r, W2, al2r, ar2r, out_sel)
    h2tabs, tab2, gm2 = list(h2c[:4]), h2c[4], h2c[5]
    ehat2, s2 = _pass_a(src, dst, tab2, gm2, z16)
    rs2 = _recip(s2)
    heads_of2 = [[2 * c + jj // 4 for jj in range(8)] for c in range(4)]
    p2 = _pass_b(src, dst, ehat2, rs2, h2tabs, z128, heads_of=heads_of2)

    parts = []
    for c in range(4):
        parts.append(p2[c][0])
        parts.append(p2[c][1])
    return _final(parts, b2c, mean_m)


# speedup vs baseline: 1.0182x; 1.0182x over previous
import functools

import jax
import jax.numpy as jnp
from jax import lax
from jax.experimental import pallas as pl
from jax.experimental.pallas import tpu as pltpu
from jax.experimental.pallas import tpu_sc as plsc

N = 10000
E = 320000
IN_SIZE = 128
HID = 16
OUT_SIZE = 64
HEADS = 8

NC = 2
NS = 16
NW = NC * NS
EPT = E // NW
RB = 624
TAIL0 = NS * RB
TAILN = N - TAIL0
KA = 1000
KB = 200
NCH = EPT // KB

_f32 = jnp.float32
_i32 = jnp.int32


def _vgather(x, idx):
    return lax.gather(
        x, idx[:, None],
        dimension_numbers=lax.GatherDimensionNumbers(
            offset_dims=(), collapsed_slice_dims=(0,), start_index_map=(0,)),
        slice_sizes=(1,),
        mode=lax.GatherScatterMode.PROMISE_IN_BOUNDS)


_MESH = plsc.VectorSubcoreMesh(
    core_axis_name="c", subcore_axis_name="s",
    num_cores=NC, num_subcores=NS)
_SC_PARAMS = pltpu.CompilerParams(use_tc_tiling_on_sc=False)


def _dense1(x, W1, al1, ar1, S8):
    R = 1000
    grid = N // R

    def body(x_r, w_r, al_r, ar_r, s8_r, h0_r, h1_r, tab_r, gm_r):
        pid = pl.program_id(0)
        h = jnp.dot(x_r[...], w_r[...], preferred_element_type=_f32)
        h0_r[...] = h[:, 0:64]
        h1_r[...] = h[:, 64:128]
        el = jnp.dot(h * al_r[...], s8_r[...], preferred_element_type=_f32)
        er = jnp.dot(h * ar_r[...], s8_r[...], preferred_element_type=_f32)
        tab_r[...] = jnp.concatenate([el, er], axis=1)
        m = jnp.max(el, axis=0, keepdims=True)
        rowb = jnp.broadcast_to(jnp.concatenate([m, m], axis=1), (8, 16))

        @pl.when(pid == 0)
        def _():
            gm_r[...] = rowb

        @pl.when(pid != 0)
        def _():
            gm_r[...] = jnp.maximum(gm_r[...], rowb)

    return pl.pallas_call(
        body,
        grid=(grid,),
        in_specs=[
            pl.BlockSpec((R, IN_SIZE), lambda i: (i, 0)),
            pl.BlockSpec((IN_SIZE, IN_SIZE), lambda i: (0, 0)),
            pl.BlockSpec((1, IN_SIZE), lambda i: (0, 0)),
            pl.BlockSpec((1, IN_SIZE), lambda i: (0, 0)),
            pl.BlockSpec((IN_SIZE, 8), lambda i: (0, 0)),
        ],
        out_specs=[
            pl.BlockSpec((R, 64), lambda i: (i, 0)),
            pl.BlockSpec((R, 64), lambda i: (i, 0)),
            pl.BlockSpec((R, 16), lambda i: (i, 0)),
            pl.BlockSpec((8, 16), lambda i: (0, 0)),
        ],
        out_shape=[
            jax.ShapeDtypeStruct((N, 64), _f32),
            jax.ShapeDtypeStruct((N, 64), _f32),
            jax.ShapeDtypeStruct((N, 16), _f32),
            jax.ShapeDtypeStruct((8, 16), _f32),
        ],
    )(x, W1, al1, ar1, S8)


def _pass_a(src, dst, tab, gmax, z16):
    @functools.partial(
        pl.kernel,
        out_type=[jax.ShapeDtypeStruct((E, 16), _f32),
                  jax.ShapeDtypeStruct((NC, N, 16), _f32)],
        mesh=_MESH,
        compiler_params=_SC_PARAMS,
        scratch_types=[
            pltpu.VMEM((KA,), _i32), pltpu.VMEM((KA,), _i32),
            pltpu.VMEM((KA, 16), _f32), pltpu.VMEM((KA, 16), _f32),
            pltpu.VMEM((KA, 16), _f32), pltpu.VMEM((16,), _f32),
            pltpu.VMEM_SHARED((N, 16), _f32),
            pltpu.SemaphoreType.DMA, pltpu.SemaphoreType.DMA,
        ],
    )
    def k(src_h, dst_h, tab_h, gm_h, z_h, ehat_h, s_h,
          srcb, dstb, srows, drows, ebuf, gbuf, sacc, sem1, sem2):
        cid = lax.axis_index("c")
        sid = lax.axis_index("s")
        w = sid * NC + cid
        r0 = sid * RB
        pltpu.sync_copy(z_h.at[pl.ds(r0, RB)], sacc.at[pl.ds(r0, RB)])

        @pl.when(sid == 0)
        def _():
            pltpu.sync_copy(z_h.at[pl.ds(TAIL0, TAILN)],
                            sacc.at[pl.ds(TAIL0, TAILN)])

        pltpu.sync_copy(gm_h.at[0], gbuf)
        plsc.subcore_barrier()

        lanes = lax.broadcasted_iota(_i32, (16,), 0)
        sel = lanes < 8
        i07 = lax.bitwise_and(lanes, 7)
        i7p8 = i07 + 8
        gv = gbuf[...]

        def chunk(j, carry):
            off = w * EPT + j * KA
            pltpu.sync_copy(src_h.at[pl.ds(off, KA)], srcb)
            pltpu.sync_copy(dst_h.at[pl.ds(off, KA)], dstb)
            cp1 = pltpu.async_copy(tab_h.at[srcb], srows, sem1)
            cp2 = pltpu.async_copy(tab_h.at[dstb], drows, sem2)
            cp1.wait()
            cp2.wait()

            def pair(i, c2):
                s0 = srows[2 * i]
                s1 = srows[2 * i + 1]
                d0 = drows[2 * i]
                d1 = drows[2 * i + 1]
                el2 = jnp.where(sel, s0, _vgather(s1, i07))
                er2 = jnp.where(sel, _vgather(d0, i7p8), d1)
                z = el2 + er2
                e = jnp.where(z >= 0.0, z, 0.2 * z)
                mh = jnp.maximum(gv + er2, 0.0)
                eh = jnp.exp(e - mh)
                ebuf[2 * i] = jnp.where(sel, eh, _vgather(eh, i07))
                ebuf[2 * i + 1] = jnp.where(sel, _vgather(eh, i7p8), eh)
                return c2

            lax.fori_loop(0, KA // 2, pair, 0)
            pltpu.sync_copy(ebuf, ehat_h.at[pl.ds(off, KA)])
            pltpu.sync_copy(ebuf, sacc.at[dstb], add=True)
            return carry

        lax.fori_loop(0, EPT // KA, chunk, 0)
        plsc.subcore_barrier()
        pltpu.sync_copy(sacc.at[pl.ds(r0, RB)],
                        s_h.at[cid, pl.ds(r0, RB)])

        @pl.when(sid == 0)
        def _():
            pltpu.sync_copy(sacc.at[pl.ds(TAIL0, TAILN)],
                            s_h.at[cid, pl.ds(TAIL0, TAILN)])

    return k(src, dst, tab, gmax, z16)


def _recip(s):
    R = 1000
    grid = N // R

    def body(a_r, b_r, o_r):
        o_r[...] = 1.0 / (a_r[...] + b_r[...] + 1e-9)

    return pl.pallas_call(
        body,
        grid=(grid,),
        in_specs=[pl.BlockSpec((R, 16), lambda i: (i, 0)),
                  pl.BlockSpec((R, 16), lambda i: (i, 0))],
        out_specs=pl.BlockSpec((R, 16), lambda i: (i, 0)),
        out_shape=jax.ShapeDtypeStruct((N, 16), _f32),
    )(s[0], s[1])


def _pass_b(src, dst, ehat, rs, tables, z64, heads_of):
    C = len(tables)

    @functools.partial(
        pl.kernel,
        out_type=[jax.ShapeDtypeStruct((NC, N, 64), _f32) for _ in range(C)],
        mesh=_MESH,
        compiler_params=_SC_PARAMS,
        scratch_types=[
            [pltpu.VMEM((KB,), _i32)] * 2,
            [pltpu.VMEM((KB,), _i32)] * 2,
            [pltpu.VMEM((KB, 64), _f32)] * 2,
            [pltpu.VMEM((KB, 16), _f32)] * 2,
            [pltpu.VMEM((KB, 16), _f32)] * 2,
            pltpu.VMEM((KB, 64), _f32),
            pltpu.VMEM_SHARED((N, 64), _f32),
            [pltpu.SemaphoreType.DMA] * 2,
            [pltpu.SemaphoreType.DMA] * 2,
        ],
    )
    def k(src_h, dst_h, ehat_h, rs_h, *rest):
        tabs = rest[:C]
        z_h = rest[C]
        outs = rest[C + 1:2 * C + 1]
        (srcb, dstb, hrows, ebuf, rsrows, msgb, acc, semh, semr) = \
            rest[2 * C + 1:]
        cid = lax.axis_index("c")
        sid = lax.axis_index("s")
        w = sid * NC + cid
        r0 = sid * RB
        base = w * EPT
        splats = [jnp.full((16,), h, _i32) for h in range(HEADS)]

        for c in range(C):
            def stage(ch, b, _c=c):
                off = base + ch * KB
                pltpu.sync_copy(src_h.at[pl.ds(off, KB)], srcb[b])
                pltpu.sync_copy(dst_h.at[pl.ds(off, KB)], dstb[b])
                pltpu.sync_copy(ehat_h.at[pl.ds(off, KB)], ebuf[b])
                pltpu.async_copy(tabs[_c].at[srcb[b]], hrows[b], semh[b])
                pltpu.async_copy(rs_h.at[dstb[b]], rsrows[b], semr[b])

            pltpu.sync_copy(z_h.at[pl.ds(r0, RB)], acc.at[pl.ds(r0, RB)])

            @pl.when(sid == 0)
            def _():
                pltpu.sync_copy(z_h.at[pl.ds(TAIL0, TAILN)],
                                acc.at[pl.ds(TAIL0, TAILN)])

            plsc.subcore_barrier()

            for b in range(2):
                stage(b, b)

            def jloop(jj, carry, _c=c):
                for b in range(2):
                    ch = 2 * jj + b
                    pltpu.make_async_copy(
                        tabs[_c].at[srcb[b]], hrows[b], semh[b]).wait()
                    pltpu.make_async_copy(
                        rs_h.at[dstb[b]], rsrows[b], semr[b]).wait()

                    def edge(kk, c2, _b=b, _cc=_c):
                        arow = ebuf[_b][kk] * rsrows[_b][kk]
                        for g in range(4):
                            sp = _vgather(arow, splats[heads_of[_cc][g]])
                            msgb[kk, pl.ds(16 * g, 16)] = (
                                hrows[_b][kk, pl.ds(16 * g, 16)] * sp)
                        return c2

                    lax.fori_loop(0, KB, edge, 0)
                    pltpu.sync_copy(msgb, acc.at[dstb[b]], add=True)

                    @pl.when(ch + 2 < NCH)
                    def _(_b=b, _ch=ch):
                        stage(_ch + 2, _b)
                return carry

            lax.fori_loop(0, NCH // 2, jloop, 0)
            plsc.subcore_barrier()
            pltpu.sync_copy(acc.at[pl.ds(r0, RB)],
                            outs[c].at[cid, pl.ds(r0, RB)])

            @pl.when(sid == 0)
            def _(_c=c):
                pltpu.sync_copy(acc.at[pl.ds(TAIL0, TAILN)],
                                outs[_c].at[cid, pl.ds(TAIL0, TAILN)])

    return k(src, dst, ehat, rs, *tables, z64)


def _dense2(p1parts, b1, W2, al2, ar2, S8):
    R = 1000
    grid = N // R
    F = HEADS * OUT_SIZE

    def body(q00, q01, q10, q11, b1_r, w2_r, al_r, ar_r, s8_r, *outs):
        pid = pl.program_id(0)
        h_refs = outs[:8]
        tab_r = outs[8]
        gm_r = outs[9]
        o1 = jnp.concatenate(
            [q00[...] + q01[...], q10[...] + q11[...]], axis=1)
        o1 = jnp.maximum(o1 + b1_r[...], 0.0)
        h2 = jnp.dot(o1, w2_r[...], preferred_element_type=_f32)
        for cc in range(8):
            h_refs[cc][...] = h2[:, 64 * cc:64 * cc + 64]
        el = jnp.dot(h2 * al_r[...], s8_r[...], preferred_element_type=_f32)
        er = jnp.dot(h2 * ar_r[...], s8_r[...], preferred_element_type=_f32)
        tab_r[...] = jnp.concatenate([el, er], axis=1)
        m = jnp.max(el, axis=0, keepdims=True)
        rowb = jnp.broadcast_to(jnp.concatenate([m, m], axis=1), (8, 16))

        @pl.when(pid == 0)
        def _():
            gm_r[...] = rowb

        @pl.when(pid != 0)
        def _():
            gm_r[...] = jnp.maximum(gm_r[...], rowb)

    return pl.pallas_call(
        body,
        grid=(grid,),
        in_specs=[pl.BlockSpec((R, 64), lambda i: (i, 0)) for _ in range(4)]
        + [
            pl.BlockSpec((1, 128), lambda i: (0, 0)),
            pl.BlockSpec((128, F), lambda i: (0, 0)),
            pl.BlockSpec((1, F), lambda i: (0, 0)),
            pl.BlockSpec((1, F), lambda i: (0, 0)),
            pl.BlockSpec((F, 8), lambda i: (0, 0)),
        ],
        out_specs=[pl.BlockSpec((R, 64), lambda i: (i, 0)) for _ in range(8)]
        + [pl.BlockSpec((R, 16), lambda i: (i, 0)),
           pl.BlockSpec((8, 16), lambda i: (0, 0))],
        out_shape=[jax.ShapeDtypeStruct((N, 64), _f32) for _ in range(8)]
        + [jax.ShapeDtypeStruct((N, 16), _f32),
           jax.ShapeDtypeStruct((8, 16), _f32)],
    )(*p1parts, b1, W2, al2, ar2, S8)


def _final(parts, b2sum):
    R = 1000
    grid = N // R

    def body(*refs):
        ins = refs[:16]
        bs_r = refs[16]
        o_r = refs[17]
        z = bs_r[...]
        for q in ins:
            z = z + q[...]
        z = z * (1.0 / HEADS)
        t = z - jnp.max(z, axis=1, keepdims=True)
        o_r[...] = t - jnp.log(jnp.sum(jnp.exp(t), axis=1, keepdims=True))

    return pl.pallas_call(
        body,
        grid=(grid,),
        in_specs=[pl.BlockSpec((R, 64), lambda i: (i, 0)) for _ in range(16)]
        + [pl.BlockSpec((1, 64), lambda i: (0, 0))],
        out_specs=pl.BlockSpec((R, OUT_SIZE), lambda i: (i, 0)),
        out_shape=jax.ShapeDtypeStruct((N, OUT_SIZE), _f32),
    )(*parts, b2sum)


def kernel(features, edge_index, W1, al1, ar1, b1, W2, al2, ar2, b2):
    src = edge_index[0].astype(_i32)
    dst = edge_index[1].astype(_i32)

    al1r = al1.reshape(1, HEADS * HID)
    ar1r = ar1.reshape(1, HEADS * HID)
    al2r = al2.reshape(1, HEADS * OUT_SIZE)
    ar2r = ar2.reshape(1, HEADS * OUT_SIZE)
    b1r = b1.reshape(1, HEADS * HID)
    b2sum = b2.reshape(HEADS, OUT_SIZE).sum(axis=0).reshape(1, OUT_SIZE)
    hid_sel = jnp.equal(
        jnp.arange(HEADS * HID)[:, None] // HID,
        jnp.arange(HEADS)[None, :]).astype(_f32)
    out_sel = jnp.equal(
        jnp.arange(HEADS * OUT_SIZE)[:, None] // OUT_SIZE,
        jnp.arange(HEADS)[None, :]).astype(_f32)
    z16 = jnp.zeros((N, 16), _f32)
    z64 = jnp.zeros((N, 64), _f32)

    h1a, h1b, tab1, gm1 = _dense1(features, W1, al1r, ar1r, hid_sel)
    ehat1, s1 = _pass_a(src, dst, tab1, gm1, z16)
    rs1 = _recip(s1)
    p1 = _pass_b(src, dst, ehat1, rs1, [h1a, h1b], z64,
                 heads_of=[[0, 1, 2, 3], [4, 5, 6, 7]])

    p1parts = [p1[0][0], p1[0][1], p1[1][0], p1[1][1]]
    d2 = _dense2(p1parts, b1r, W2, al2r, ar2r, out_sel)
    h2tabs, tab2, gm2 = list(d2[:8]), d2[8], d2[9]
    ehat2, s2 = _pass_a(src, dst, tab2, gm2, z16)
    rs2 = _recip(s2)
    p2 = _pass_b(src, dst, ehat2, rs2, h2tabs, z64,
                 heads_of=[[c] * 4 for c in range(8)])

    parts = []
    for c in range(8):
        parts.append(p2[c][0])
        parts.append(p2[c][1])
    return _final(parts, b2sum)


# speedup vs baseline: 2.1497x; 2.1113x over previous
import functools

import jax
import jax.numpy as jnp
from jax import lax
from jax.experimental import pallas as pl
from jax.experimental.pallas import tpu as pltpu
from jax.experimental.pallas import tpu_sc as plsc

N = 10000
E = 320000
IN_SIZE = 128
HID = 16
OUT_SIZE = 64
HEADS = 8

NC = 2
NS = 16
NW = NC * NS
EPT = E // NW
RB = 624
TAIL0 = NS * RB
TAILN = N - TAIL0
KA = 1000
KB = 200
NCH = EPT // KB

_f32 = jnp.float32
_i32 = jnp.int32


def _vgather(x, idx):
    return lax.gather(
        x, idx[:, None],
        dimension_numbers=lax.GatherDimensionNumbers(
            offset_dims=(), collapsed_slice_dims=(0,), start_index_map=(0,)),
        slice_sizes=(1,),
        mode=lax.GatherScatterMode.PROMISE_IN_BOUNDS)


_MESH = plsc.VectorSubcoreMesh(
    core_axis_name="c", subcore_axis_name="s",
    num_cores=NC, num_subcores=NS)
_SC_PARAMS = pltpu.CompilerParams(use_tc_tiling_on_sc=False)


def _dense1(x, W1, al1, ar1, S8):
    R = 1000
    grid = N // R

    def body(x_r, w_r, al_r, ar_r, s8_r, h_r, tab_r, gm_r):
        pid = pl.program_id(0)
        h = jnp.dot(x_r[...], w_r[...], preferred_element_type=_f32)
        h_r[...] = h
        el = jnp.dot(h * al_r[...], s8_r[...], preferred_element_type=_f32)
        er = jnp.dot(h * ar_r[...], s8_r[...], preferred_element_type=_f32)
        tab_r[...] = jnp.concatenate([el, er], axis=1)
        m = jnp.max(el, axis=0, keepdims=True)
        rowb = jnp.broadcast_to(jnp.concatenate([m, m], axis=1), (8, 16))

        @pl.when(pid == 0)
        def _():
            gm_r[...] = rowb

        @pl.when(pid != 0)
        def _():
            gm_r[...] = jnp.maximum(gm_r[...], rowb)

    return pl.pallas_call(
        body,
        grid=(grid,),
        in_specs=[
            pl.BlockSpec((R, IN_SIZE), lambda i: (i, 0)),
            pl.BlockSpec((IN_SIZE, IN_SIZE), lambda i: (0, 0)),
            pl.BlockSpec((1, IN_SIZE), lambda i: (0, 0)),
            pl.BlockSpec((1, IN_SIZE), lambda i: (0, 0)),
            pl.BlockSpec((IN_SIZE, 8), lambda i: (0, 0)),
        ],
        out_specs=[
            pl.BlockSpec((R, IN_SIZE), lambda i: (i, 0)),
            pl.BlockSpec((R, 16), lambda i: (i, 0)),
            pl.BlockSpec((8, 16), lambda i: (0, 0)),
        ],
        out_shape=[
            jax.ShapeDtypeStruct((N, IN_SIZE), _f32),
            jax.ShapeDtypeStruct((N, 16), _f32),
            jax.ShapeDtypeStruct((8, 16), _f32),
        ],
    )(x, W1, al1, ar1, S8)


def _pass_a(src, dst, tab, gmax, z16):
    @functools.partial(
        pl.kernel,
        out_type=[jax.ShapeDtypeStruct((E, 16), _f32),
                  jax.ShapeDtypeStruct((NC, N, 16), _f32)],
        mesh=_MESH,
        compiler_params=_SC_PARAMS,
        scratch_types=[
            pltpu.VMEM((KA,), _i32), pltpu.VMEM((KA,), _i32),
            pltpu.VMEM((KA, 16), _f32), pltpu.VMEM((KA, 16), _f32),
            pltpu.VMEM((KA, 16), _f32), pltpu.VMEM((16,), _f32),
            pltpu.VMEM_SHARED((N, 16), _f32),
            pltpu.SemaphoreType.DMA, pltpu.SemaphoreType.DMA,
        ],
    )
    def k(src_h, dst_h, tab_h, gm_h, z_h, ehat_h, s_h,
          srcb, dstb, srows, drows, ebuf, gbuf, sacc, sem1, sem2):
        cid = lax.axis_index("c")
        sid = lax.axis_index("s")
        w = sid * NC + cid
        r0 = sid * RB
        pltpu.sync_copy(z_h.at[pl.ds(r0, RB)], sacc.at[pl.ds(r0, RB)])

        @pl.when(sid == 0)
        def _():
            pltpu.sync_copy(z_h.at[pl.ds(TAIL0, TAILN)],
                            sacc.at[pl.ds(TAIL0, TAILN)])

        pltpu.sync_copy(gm_h.at[0], gbuf)
        plsc.subcore_barrier()

        lanes = lax.broadcasted_iota(_i32, (16,), 0)
        sel = lanes < 8
        i07 = lax.bitwise_and(lanes, 7)
        i7p8 = i07 + 8
        gv = gbuf[...]

        def chunk(j, carry):
            off = w * EPT + j * KA
            pltpu.sync_copy(src_h.at[pl.ds(off, KA)], srcb)
            pltpu.sync_copy(dst_h.at[pl.ds(off, KA)], dstb)
            cp1 = pltpu.async_copy(tab_h.at[srcb], srows, sem1)
            cp2 = pltpu.async_copy(tab_h.at[dstb], drows, sem2)
            cp1.wait()
            cp2.wait()

            def pair(i, c2):
                s0 = srows[2 * i]
                s1 = srows[2 * i + 1]
                d0 = drows[2 * i]
                d1 = drows[2 * i + 1]
                el2 = jnp.where(sel, s0, _vgather(s1, i07))
                er2 = jnp.where(sel, _vgather(d0, i7p8), d1)
                z = el2 + er2
                e = jnp.where(z >= 0.0, z, 0.2 * z)
                mh = jnp.maximum(gv + er2, 0.0)
                eh = jnp.exp(e - mh)
                ebuf[2 * i] = jnp.where(sel, eh, _vgather(eh, i07))
                ebuf[2 * i + 1] = jnp.where(sel, _vgather(eh, i7p8), eh)
                return c2

            lax.fori_loop(0, KA // 2, pair, 0)
            pltpu.sync_copy(ebuf, ehat_h.at[pl.ds(off, KA)])
            pltpu.sync_copy(ebuf, sacc.at[dstb], add=True)
            return carry

        lax.fori_loop(0, EPT // KA, chunk, 0)
        plsc.subcore_barrier()
        pltpu.sync_copy(sacc.at[pl.ds(r0, RB)],
                        s_h.at[cid, pl.ds(r0, RB)])

        @pl.when(sid == 0)
        def _():
            pltpu.sync_copy(sacc.at[pl.ds(TAIL0, TAILN)],
                            s_h.at[cid, pl.ds(TAIL0, TAILN)])

    return k(src, dst, tab, gmax, z16)


def _recip(s):
    R = 1000
    grid = N // R

    def body(a_r, b_r, o_r):
        o_r[...] = 1.0 / (a_r[...] + b_r[...] + 1e-9)

    return pl.pallas_call(
        body,
        grid=(grid,),
        in_specs=[pl.BlockSpec((R, 16), lambda i: (i, 0)),
                  pl.BlockSpec((R, 16), lambda i: (i, 0))],
        out_specs=pl.BlockSpec((R, 16), lambda i: (i, 0)),
        out_shape=jax.ShapeDtypeStruct((N, 16), _f32),
    )(s[0], s[1])


def _pass_b(src, dst, ehat, tables, z128, heads_of):
    C = len(tables)

    @functools.partial(
        pl.kernel,
        out_type=[jax.ShapeDtypeStruct((NC, N, 128), _f32) for _ in range(C)],
        mesh=_MESH,
        compiler_params=_SC_PARAMS,
        scratch_types=[
            pltpu.VMEM((KB,), _i32), pltpu.VMEM((KB,), _i32),
            pltpu.VMEM((KB + 8, 128), _f32), pltpu.VMEM((KB + 8, 16), _f32),
            pltpu.VMEM_SHARED((N, 128), _f32),
            pltpu.SemaphoreType.DMA, pltpu.SemaphoreType.DMA,
        ],
    )
    def k(src_h, dst_h, ehat_h, *rest):
        tabs = rest[:C]
        z_h = rest[C]
        outs = rest[C + 1:2 * C + 1]
        (srcb, dstb, hrows, ebuf, acc, sem1, sem2) = rest[2 * C + 1:]
        cid = lax.axis_index("c")
        sid = lax.axis_index("s")
        w = sid * NC + cid
        r0 = sid * RB
        base = w * EPT
        splats = [jnp.full((16,), h, _i32) for h in range(HEADS)]

        for c in range(C):
            heads = heads_of[c]
            uheads = sorted(set(heads))

            pltpu.sync_copy(z_h.at[pl.ds(r0, RB)], acc.at[pl.ds(r0, RB)])

            @pl.when(sid == 0)
            def _():
                pltpu.sync_copy(z_h.at[pl.ds(TAIL0, TAILN)],
                                acc.at[pl.ds(TAIL0, TAILN)])

            plsc.subcore_barrier()

            def echunk(j, carry, _c=c, _heads=heads, _uheads=uheads):
                off = base + j * KB
                pltpu.sync_copy(src_h.at[pl.ds(off, KB)], srcb)
                pltpu.sync_copy(dst_h.at[pl.ds(off, KB)], dstb)
                cph = pltpu.async_copy(
                    tabs[_c].at[srcb], hrows.at[pl.ds(0, KB)], sem1)
                cpe = pltpu.async_copy(
                    ehat_h.at[pl.ds(off, KB)], ebuf.at[pl.ds(0, KB)], sem2)
                cph.wait()
                cpe.wait()

                init = tuple(hrows[0, pl.ds(16 * g, 16)] for g in range(8)
                             ) + (ebuf[0],)

                def edge(kk, cr, _hh=_heads, _uu=_uheads):
                    hv = cr[:8]
                    erow = cr[8]
                    sp = {h: _vgather(erow, splats[h]) for h in _uu}
                    for g in range(8):
                        hrows[kk, pl.ds(16 * g, 16)] = hv[g] * sp[_hh[g]]
                    nxt = tuple(hrows[kk + 1, pl.ds(16 * g, 16)]
                                for g in range(8))
                    return nxt + (ebuf[kk + 1],)

                lax.fori_loop(0, KB, edge, init)
                pltpu.sync_copy(hrows.at[pl.ds(0, KB)], acc.at[dstb],
                                add=True)
                return carry

            lax.fori_loop(0, NCH, echunk, 0)
            plsc.subcore_barrier()
            pltpu.sync_copy(acc.at[pl.ds(r0, RB)],
                            outs[c].at[cid, pl.ds(r0, RB)])

            @pl.when(sid == 0)
            def _(_c=c):
                pltpu.sync_copy(acc.at[pl.ds(TAIL0, TAILN)],
                                outs[_c].at[cid, pl.ds(TAIL0, TAILN)])

    return k(src, dst, ehat, *tables, z128)


def _dense2(p0, p1, rs1, EXP1, b1, W2, al2, ar2, S8):
    R = 1000
    grid = N // R
    F = HEADS * OUT_SIZE

    def body(q0, q1, rs_r, e1_r, b1_r, w2_r, al_r, ar_r, s8_r, *outs):
        pid = pl.program_id(0)
        h0_r, h1_r, h2_r, h3_r, tab_r, gm_r = outs
        rse = jnp.dot(rs_r[...], e1_r[...], preferred_element_type=_f32)
        o1 = (q0[...] + q1[...]) * rse
        o1 = jnp.maximum(o1 + b1_r[...], 0.0)
        h2 = jnp.dot(o1, w2_r[...], preferred_element_type=_f32)
        h0_r[...] = h2[:, 0:128]
        h1_r[...] = h2[:, 128:256]
        h2_r[...] = h2[:, 256:384]
        h3_r[...] = h2[:, 384:512]
        el = jnp.dot(h2 * al_r[...], s8_r[...], preferred_element_type=_f32)
        er = jnp.dot(h2 * ar_r[...], s8_r[...], preferred_element_type=_f32)
        tab_r[...] = jnp.concatenate([el, er], axis=1)
        m = jnp.max(el, axis=0, keepdims=True)
        rowb = jnp.broadcast_to(jnp.concatenate([m, m], axis=1), (8, 16))

        @pl.when(pid == 0)
        def _():
            gm_r[...] = rowb

        @pl.when(pid != 0)
        def _():
            gm_r[...] = jnp.maximum(gm_r[...], rowb)

    return pl.pallas_call(
        body,
        grid=(grid,),
        in_specs=[
            pl.BlockSpec((R, 128), lambda i: (i, 0)),
            pl.BlockSpec((R, 128), lambda i: (i, 0)),
            pl.BlockSpec((R, 16), lambda i: (i, 0)),
            pl.BlockSpec((16, 128), lambda i: (0, 0)),
            pl.BlockSpec((1, 128), lambda i: (0, 0)),
            pl.BlockSpec((128, F), lambda i: (0, 0)),
            pl.BlockSpec((1, F), lambda i: (0, 0)),
            pl.BlockSpec((1, F), lambda i: (0, 0)),
            pl.BlockSpec((F, 8), lambda i: (0, 0)),
        ],
        out_specs=[pl.BlockSpec((R, 128), lambda i: (i, 0)) for _ in range(4)]
        + [pl.BlockSpec((R, 16), lambda i: (i, 0)),
           pl.BlockSpec((8, 16), lambda i: (0, 0))],
        out_shape=[jax.ShapeDtypeStruct((N, 128), _f32) for _ in range(4)]
        + [jax.ShapeDtypeStruct((N, 16), _f32),
           jax.ShapeDtypeStruct((8, 16), _f32)],
    )(p0, p1, rs1, EXP1, b1, W2, al2, ar2, S8)


def _final(parts, rs2, EXP2, M128, b2mean):
    R = 1000
    grid = N // R

    def body(*refs):
        ins = refs[:8]
        rs_r, e2_r, m_r, bm_r, o_r = refs[8:]
        z = jnp.broadcast_to(bm_r[...], (R, OUT_SIZE))
        for c in range(4):
            rse = jnp.dot(rs_r[...], e2_r[pl.ds(16 * c, 16), :],
                          preferred_element_type=_f32)
            t = (ins[2 * c][...] + ins[2 * c + 1][...]) * rse
            z = z + jnp.dot(t, m_r[...], preferred_element_type=_f32)
        t = z - jnp.max(z, axis=1, keepdims=True)
        o_r[...] = t - jnp.log(jnp.sum(jnp.exp(t), axis=1, keepdims=True))

    return pl.pallas_call(
        body,
        grid=(grid,),
        in_specs=[pl.BlockSpec((R, 128), lambda i: (i, 0)) for _ in range(8)]
        + [pl.BlockSpec((R, 16), lambda i: (i, 0)),
           pl.BlockSpec((64, 128), lambda i: (0, 0)),
           pl.BlockSpec((128, OUT_SIZE), lambda i: (0, 0)),
           pl.BlockSpec((1, OUT_SIZE), lambda i: (0, 0))],
        out_specs=pl.BlockSpec((R, OUT_SIZE), lambda i: (i, 0)),
        out_shape=jax.ShapeDtypeStruct((N, OUT_SIZE), _f32),
    )(*parts, rs2, EXP2, M128, b2mean)


def kernel(features, edge_index, W1, al1, ar1, b1, W2, al2, ar2, b2):
    src = edge_index[0].astype(_i32)
    dst = edge_index[1].astype(_i32)

    al1r = al1.reshape(1, HEADS * HID)
    ar1r = ar1.reshape(1, HEADS * HID)
    al2r = al2.reshape(1, HEADS * OUT_SIZE)
    ar2r = ar2.reshape(1, HEADS * OUT_SIZE)
    b1r = b1.reshape(1, HEADS * HID)
    b2mean = b2.reshape(HEADS, OUT_SIZE).mean(axis=0).reshape(1, OUT_SIZE)
    hid_sel = jnp.equal(
        jnp.arange(HEADS * HID)[:, None] // HID,
        jnp.arange(HEADS)[None, :]).astype(_f32)
    out_sel = jnp.equal(
        jnp.arange(HEADS * OUT_SIZE)[:, None] // OUT_SIZE,
        jnp.arange(HEADS)[None, :]).astype(_f32)
    exp1 = jnp.equal(
        jnp.arange(16)[:, None],
        jnp.arange(128)[None, :] // HID).astype(_f32)
    exp2 = jnp.concatenate([
        jnp.equal(jnp.arange(16)[:, None],
                  2 * c + jnp.arange(128)[None, :] // OUT_SIZE).astype(_f32)
        for c in range(4)], axis=0)
    m128 = jnp.tile(jnp.eye(OUT_SIZE, dtype=_f32), (2, 1)) / HEADS
    z16 = jnp.zeros((N, 16), _f32)
    z128 = jnp.zeros((N, 128), _f32)

    h1, tab1, gm1 = _dense1(features, W1, al1r, ar1r, hid_sel)
    ehat1, s1 = _pass_a(src, dst, tab1, gm1, z16)
    rs1 = _recip(s1)
    (p1,) = _pass_b(src, dst, ehat1, [h1], z128,
                    heads_of=[list(range(8))])

    d2 = _dense2(p1[0], p1[1], rs1, exp1, b1r, W2, al2r, ar2r, out_sel)
    h2tabs, tab2, gm2 = list(d2[:4]), d2[4], d2[5]
    ehat2, s2 = _pass_a(src, dst, tab2, gm2, z16)
    rs2 = _recip(s2)
    heads_of2 = [[2 * c + g // 4 for g in range(8)] for c in range(4)]
    p2 = _pass_b(src, dst, ehat2, h2tabs, z128, heads_of=heads_of2)

    parts = []
    for c in range(4):
        parts.append(p2[c][0])
        parts.append(p2[c][1])
    return _final(parts, rs2, exp2, m128, b2mean)


# speedup vs baseline: 2.6929x; 1.2527x over previous
import functools

import jax
import jax.numpy as jnp
from jax import lax
from jax.experimental import pallas as pl
from jax.experimental.pallas import tpu as pltpu
from jax.experimental.pallas import tpu_sc as plsc

N = 10000
E = 320000
IN_SIZE = 128
HID = 16
OUT_SIZE = 64
HEADS = 8

NC = 2
NS = 16
NW = NC * NS
EPT = E // NW
RB = 624
TAIL0 = NS * RB
TAILN = N - TAIL0
KA = 1000
KB = 200
NCH = EPT // KB

_f32 = jnp.float32
_i32 = jnp.int32


def _vgather(x, idx):
    return lax.gather(
        x, idx[:, None],
        dimension_numbers=lax.GatherDimensionNumbers(
            offset_dims=(), collapsed_slice_dims=(0,), start_index_map=(0,)),
        slice_sizes=(1,),
        mode=lax.GatherScatterMode.PROMISE_IN_BOUNDS)


_MESH = plsc.VectorSubcoreMesh(
    core_axis_name="c", subcore_axis_name="s",
    num_cores=NC, num_subcores=NS)
_SC_PARAMS = pltpu.CompilerParams(use_tc_tiling_on_sc=False)


def _dense1(x, W1, al1, ar1, S8):
    R = 1000
    grid = N // R

    def body(x_r, w_r, al_r, ar_r, s8_r, h_r, tab_r, gm_r):
        pid = pl.program_id(0)
        h = jnp.dot(x_r[...], w_r[...], preferred_element_type=_f32)
        h_r[...] = h
        el = jnp.dot(h * al_r[...], s8_r[...], preferred_element_type=_f32)
        er = jnp.dot(h * ar_r[...], s8_r[...], preferred_element_type=_f32)
        tab_r[...] = jnp.concatenate([el, er], axis=1)
        m = jnp.max(el, axis=0, keepdims=True)
        rowb = jnp.broadcast_to(jnp.concatenate([m, m], axis=1), (8, 16))

        @pl.when(pid == 0)
        def _():
            gm_r[...] = rowb

        @pl.when(pid != 0)
        def _():
            gm_r[...] = jnp.maximum(gm_r[...], rowb)

    return pl.pallas_call(
        body,
        grid=(grid,),
        in_specs=[
            pl.BlockSpec((R, IN_SIZE), lambda i: (i, 0)),
            pl.BlockSpec((IN_SIZE, IN_SIZE), lambda i: (0, 0)),
            pl.BlockSpec((1, IN_SIZE), lambda i: (0, 0)),
            pl.BlockSpec((1, IN_SIZE), lambda i: (0, 0)),
            pl.BlockSpec((IN_SIZE, 8), lambda i: (0, 0)),
        ],
        out_specs=[
            pl.BlockSpec((R, IN_SIZE), lambda i: (i, 0)),
            pl.BlockSpec((R, 16), lambda i: (i, 0)),
            pl.BlockSpec((8, 16), lambda i: (0, 0)),
        ],
        out_shape=[
            jax.ShapeDtypeStruct((N, IN_SIZE), _f32),
            jax.ShapeDtypeStruct((N, 16), _f32),
            jax.ShapeDtypeStruct((8, 16), _f32),
        ],
    )(x, W1, al1, ar1, S8)


def _pass_a(src, dst, tab, gmax, z16):
    @functools.partial(
        pl.kernel,
        out_type=[jax.ShapeDtypeStruct((E, 16), _f32),
                  jax.ShapeDtypeStruct((NC, N, 16), _f32)],
        mesh=_MESH,
        compiler_params=_SC_PARAMS,
        scratch_types=[
            pltpu.VMEM((KA,), _i32), pltpu.VMEM((KA,), _i32),
            pltpu.VMEM((KA, 16), _f32), pltpu.VMEM((KA, 16), _f32),
            pltpu.VMEM((KA, 16), _f32), pltpu.VMEM((16,), _f32),
            pltpu.VMEM_SHARED((N, 16), _f32),
            pltpu.SemaphoreType.DMA, pltpu.SemaphoreType.DMA,
        ],
    )
    def k(src_h, dst_h, tab_h, gm_h, z_h, ehat_h, s_h,
          srcb, dstb, srows, drows, ebuf, gbuf, sacc, sem1, sem2):
        cid = lax.axis_index("c")
        sid = lax.axis_index("s")
        w = sid * NC + cid
        r0 = sid * RB
        pltpu.sync_copy(z_h.at[pl.ds(r0, RB)], sacc.at[pl.ds(r0, RB)])

        @pl.when(sid == 0)
        def _():
            pltpu.sync_copy(z_h.at[pl.ds(TAIL0, TAILN)],
                            sacc.at[pl.ds(TAIL0, TAILN)])

        pltpu.sync_copy(gm_h.at[0], gbuf)
        plsc.subcore_barrier()

        lanes = lax.broadcasted_iota(_i32, (16,), 0)
        sel = lanes < 8
        i07 = lax.bitwise_and(lanes, 7)
        i7p8 = i07 + 8
        gv = gbuf[...]

        def chunk(j, carry):
            off = w * EPT + j * KA
            pltpu.sync_copy(src_h.at[pl.ds(off, KA)], srcb)
            pltpu.sync_copy(dst_h.at[pl.ds(off, KA)], dstb)
            cp1 = pltpu.async_copy(tab_h.at[srcb], srows, sem1)
            cp2 = pltpu.async_copy(tab_h.at[dstb], drows, sem2)
            cp1.wait()
            cp2.wait()

            def pair(i, c2):
                s0 = srows[2 * i]
                s1 = srows[2 * i + 1]
                d0 = drows[2 * i]
                d1 = drows[2 * i + 1]
                el2 = jnp.where(sel, s0, _vgather(s1, i07))
                er2 = jnp.where(sel, _vgather(d0, i7p8), d1)
                z = el2 + er2
                e = jnp.where(z >= 0.0, z, 0.2 * z)
                mh = jnp.maximum(gv + er2, 0.0)
                eh = jnp.exp(e - mh)
                ebuf[2 * i] = jnp.where(sel, eh, _vgather(eh, i07))
                ebuf[2 * i + 1] = jnp.where(sel, _vgather(eh, i7p8), eh)
                return c2

            lax.fori_loop(0, KA // 2, pair, 0)
            pltpu.sync_copy(ebuf, ehat_h.at[pl.ds(off, KA)])
            pltpu.sync_copy(ebuf, sacc.at[dstb], add=True)
            return carry

        lax.fori_loop(0, EPT // KA, chunk, 0)
        plsc.subcore_barrier()
        pltpu.sync_copy(sacc.at[pl.ds(r0, RB)],
                        s_h.at[cid, pl.ds(r0, RB)])

        @pl.when(sid == 0)
        def _():
            pltpu.sync_copy(sacc.at[pl.ds(TAIL0, TAILN)],
                            s_h.at[cid, pl.ds(TAIL0, TAILN)])

    return k(src, dst, tab, gmax, z16)


def _recip(s):
    R = 1000
    grid = N // R

    def body(a_r, b_r, o_r):
        o_r[...] = 1.0 / (a_r[...] + b_r[...] + 1e-9)

    return pl.pallas_call(
        body,
        grid=(grid,),
        in_specs=[pl.BlockSpec((R, 16), lambda i: (i, 0)),
                  pl.BlockSpec((R, 16), lambda i: (i, 0))],
        out_specs=pl.BlockSpec((R, 16), lambda i: (i, 0)),
        out_shape=jax.ShapeDtypeStruct((N, 16), _f32),
    )(s[0], s[1])


def _pass_b(src, dst, ehat, tables, z128, heads_of):
    C = len(tables)
    KF = 160
    NF = EPT // KF
    KT = EPT - NF * KF

    @functools.partial(
        pl.kernel,
        out_type=[jax.ShapeDtypeStruct((NC, N, 128), _f32) for _ in range(C)],
        mesh=_MESH,
        compiler_params=_SC_PARAMS,
        scratch_types=[
            [pltpu.VMEM((KF,), _i32)] * 2,
            [pltpu.VMEM((KF,), _i32)] * 2,
            [pltpu.VMEM((KF + 8, 128), _f32)] * 2,
            [pltpu.VMEM((KF + 8, 16), _f32)] * 2,
            pltpu.VMEM_SHARED((N, 128), _f32),
            [pltpu.SemaphoreType.DMA] * 2,
            [pltpu.SemaphoreType.DMA] * 2,
            pltpu.VMEM((80,), _i32), pltpu.VMEM((80,), _i32),
        ],
    )
    def k(src_h, dst_h, ehat_h, *rest):
        tabs = rest[:C]
        z_h = rest[C]
        outs = rest[C + 1:2 * C + 1]
        (srcb, dstb, hrows, ebuf, acc, semh, seme, srct, dstt) = \
            rest[2 * C + 1:]
        cid = lax.axis_index("c")
        sid = lax.axis_index("s")
        w = sid * NC + cid
        r0 = sid * RB
        base = w * EPT
        splats = [jnp.full((16,), h, _i32) for h in range(HEADS)]

        def run_edges(b, length, _heads, _uheads):
            init = tuple(hrows[b][0, pl.ds(16 * g, 16)] for g in range(8)
                         ) + (ebuf[b][0],)

            def edge(kk, cr):
                hv = cr[:8]
                erow = cr[8]
                sp = {h: _vgather(erow, splats[h]) for h in _uheads}
                for g in range(8):
                    hrows[b][kk, pl.ds(16 * g, 16)] = hv[g] * sp[_heads[g]]
                nxt = tuple(hrows[b][kk + 1, pl.ds(16 * g, 16)]
                            for g in range(8))
                return nxt + (ebuf[b][kk + 1],)

            lax.fori_loop(0, length, edge, init)

        for c in range(C):
            heads = heads_of[c]
            uheads = sorted(set(heads))

            pltpu.sync_copy(z_h.at[pl.ds(r0, RB)], acc.at[pl.ds(r0, RB)])

            @pl.when(sid == 0)
            def _():
                pltpu.sync_copy(z_h.at[pl.ds(TAIL0, TAILN)],
                                acc.at[pl.ds(TAIL0, TAILN)])

            plsc.subcore_barrier()

            def stage(ch, b, _c=c):
                off = base + ch * KF
                pltpu.sync_copy(src_h.at[pl.ds(off, KF)], srcb[b])
                pltpu.sync_copy(dst_h.at[pl.ds(off, KF)], dstb[b])
                pltpu.async_copy(
                    tabs[_c].at[srcb[b]], hrows[b].at[pl.ds(0, KF)], semh[b])
                pltpu.async_copy(
                    ehat_h.at[pl.ds(off, KF)], ebuf[b].at[pl.ds(0, KF)],
                    seme[b])

            for b in range(2):
                stage(b, b)

            def jloop(jj, carry, _c=c, _heads=heads, _uheads=uheads):
                for b in range(2):
                    ch = 2 * jj + b
                    pltpu.make_async_copy(
                        tabs[_c].at[srcb[b]], hrows[b].at[pl.ds(0, KF)],
                        semh[b]).wait()
                    pltpu.make_async_copy(
                        ehat_h.at[pl.ds(0, KF)], ebuf[b].at[pl.ds(0, KF)],
                        seme[b]).wait()
                    run_edges(b, KF, _heads, _uheads)
                    pltpu.sync_copy(hrows[b].at[pl.ds(0, KF)],
                                    acc.at[dstb[b]], add=True)

                    @pl.when(ch + 2 < NF)
                    def _(_b=b, _ch=ch):
                        stage(_ch + 2, _b)
                return carry

            lax.fori_loop(0, NF // 2, jloop, 0)

            offt = base + NF * KF
            pltpu.sync_copy(src_h.at[pl.ds(offt, KT)], srct)
            pltpu.sync_copy(dst_h.at[pl.ds(offt, KT)], dstt)
            pltpu.async_copy(tabs[c].at[srct],
                             hrows[0].at[pl.ds(0, KT)], semh[0]).wait()
            pltpu.async_copy(ehat_h.at[pl.ds(offt, KT)],
                             ebuf[0].at[pl.ds(0, KT)], seme[0]).wait()
            run_edges(0, KT, heads, uheads)
            pltpu.sync_copy(hrows[0].at[pl.ds(0, KT)],
                            acc.at[dstt], add=True)

            plsc.subcore_barrier()
            pltpu.sync_copy(acc.at[pl.ds(r0, RB)],
                            outs[c].at[cid, pl.ds(r0, RB)])

            @pl.when(sid == 0)
            def _(_c=c):
                pltpu.sync_copy(acc.at[pl.ds(TAIL0, TAILN)],
                                outs[_c].at[cid, pl.ds(TAIL0, TAILN)])

    return k(src, dst, ehat, *tables, z128)


def _dense2(p0, p1, rs1, EXP1, b1, W2, al2, ar2, S8):
    R = 1000
    grid = N // R
    F = HEADS * OUT_SIZE

    def body(q0, q1, rs_r, e1_r, b1_r, w2_r, al_r, ar_r, s8_r, *outs):
        pid = pl.program_id(0)
        h0_r, h1_r, h2_r, h3_r, tab_r, gm_r = outs
        rse = jnp.dot(rs_r[...], e1_r[...], preferred_element_type=_f32)
        o1 = (q0[...] + q1[...]) * rse
        o1 = jnp.maximum(o1 + b1_r[...], 0.0)
        h2 = jnp.dot(o1, w2_r[...], preferred_element_type=_f32)
        h0_r[...] = h2[:, 0:128]
        h1_r[...] = h2[:, 128:256]
        h2_r[...] = h2[:, 256:384]
        h3_r[...] = h2[:, 384:512]
        el = jnp.dot(h2 * al_r[...], s8_r[...], preferred_element_type=_f32)
        er = jnp.dot(h2 * ar_r[...], s8_r[...], preferred_element_type=_f32)
        tab_r[...] = jnp.concatenate([el, er], axis=1)
        m = jnp.max(el, axis=0, keepdims=True)
        rowb = jnp.broadcast_to(jnp.concatenate([m, m], axis=1), (8, 16))

        @pl.when(pid == 0)
        def _():
            gm_r[...] = rowb

        @pl.when(pid != 0)
        def _():
            gm_r[...] = jnp.maximum(gm_r[...], rowb)

    return pl.pallas_call(
        body,
        grid=(grid,),
        in_specs=[
            pl.BlockSpec((R, 128), lambda i: (i, 0)),
            pl.BlockSpec((R, 128), lambda i: (i, 0)),
            pl.BlockSpec((R, 16), lambda i: (i, 0)),
            pl.BlockSpec((16, 128), lambda i: (0, 0)),
            pl.BlockSpec((1, 128), lambda i: (0, 0)),
            pl.BlockSpec((128, F), lambda i: (0, 0)),
            pl.BlockSpec((1, F), lambda i: (0, 0)),
            pl.BlockSpec((1, F), lambda i: (0, 0)),
            pl.BlockSpec((F, 8), lambda i: (0, 0)),
        ],
        out_specs=[pl.BlockSpec((R, 128), lambda i: (i, 0)) for _ in range(4)]
        + [pl.BlockSpec((R, 16), lambda i: (i, 0)),
           pl.BlockSpec((8, 16), lambda i: (0, 0))],
        out_shape=[jax.ShapeDtypeStruct((N, 128), _f32) for _ in range(4)]
        + [jax.ShapeDtypeStruct((N, 16), _f32),
           jax.ShapeDtypeStruct((8, 16), _f32)],
    )(p0, p1, rs1, EXP1, b1, W2, al2, ar2, S8)


def _final(parts, rs2, EXP2, M128, b2mean):
    R = 1000
    grid = N // R

    def body(*refs):
        ins = refs[:8]
        rs_r, e2_r, m_r, bm_r, o_r = refs[8:]
        z = jnp.broadcast_to(bm_r[...], (R, OUT_SIZE))
        for c in range(4):
            rse = jnp.dot(rs_r[...], e2_r[pl.ds(16 * c, 16), :],
                          preferred_element_type=_f32)
            t = (ins[2 * c][...] + ins[2 * c + 1][...]) * rse
            z = z + jnp.dot(t, m_r[...], preferred_element_type=_f32)
        t = z - jnp.max(z, axis=1, keepdims=True)
        o_r[...] = t - jnp.log(jnp.sum(jnp.exp(t), axis=1, keepdims=True))

    return pl.pallas_call(
        body,
        grid=(grid,),
        in_specs=[pl.BlockSpec((R, 128), lambda i: (i, 0)) for _ in range(8)]
        + [pl.BlockSpec((R, 16), lambda i: (i, 0)),
           pl.BlockSpec((64, 128), lambda i: (0, 0)),
           pl.BlockSpec((128, OUT_SIZE), lambda i: (0, 0)),
           pl.BlockSpec((1, OUT_SIZE), lambda i: (0, 0))],
        out_specs=pl.BlockSpec((R, OUT_SIZE), lambda i: (i, 0)),
        out_shape=jax.ShapeDtypeStruct((N, OUT_SIZE), _f32),
    )(*parts, rs2, EXP2, M128, b2mean)


def kernel(features, edge_index, W1, al1, ar1, b1, W2, al2, ar2, b2):
    src = edge_index[0].astype(_i32)
    dst = edge_index[1].astype(_i32)

    al1r = al1.reshape(1, HEADS * HID)
    ar1r = ar1.reshape(1, HEADS * HID)
    al2r = al2.reshape(1, HEADS * OUT_SIZE)
    ar2r = ar2.reshape(1, HEADS * OUT_SIZE)
    b1r = b1.reshape(1, HEADS * HID)
    b2mean = b2.reshape(HEADS, OUT_SIZE).mean(axis=0).reshape(1, OUT_SIZE)
    hid_sel = jnp.equal(
        jnp.arange(HEADS * HID)[:, None] // HID,
        jnp.arange(HEADS)[None, :]).astype(_f32)
    out_sel = jnp.equal(
        jnp.arange(HEADS * OUT_SIZE)[:, None] // OUT_SIZE,
        jnp.arange(HEADS)[None, :]).astype(_f32)
    exp1 = jnp.equal(
        jnp.arange(16)[:, None],
        jnp.arange(128)[None, :] // HID).astype(_f32)
    exp2 = jnp.concatenate([
        jnp.equal(jnp.arange(16)[:, None],
                  2 * c + jnp.arange(128)[None, :] // OUT_SIZE).astype(_f32)
        for c in range(4)], axis=0)
    m128 = jnp.tile(jnp.eye(OUT_SIZE, dtype=_f32), (2, 1)) / HEADS
    z16 = jnp.zeros((N, 16), _f32)
    z128 = jnp.zeros((N, 128), _f32)

    h1, tab1, gm1 = _dense1(features, W1, al1r, ar1r, hid_sel)
    ehat1, s1 = _pass_a(src, dst, tab1, gm1, z16)
    rs1 = _recip(s1)
    (p1,) = _pass_b(src, dst, ehat1, [h1], z128,
                    heads_of=[list(range(8))])

    d2 = _dense2(p1[0], p1[1], rs1, exp1, b1r, W2, al2r, ar2r, out_sel)
    h2tabs, tab2, gm2 = list(d2[:4]), d2[4], d2[5]
    ehat2, s2 = _pass_a(src, dst, tab2, gm2, z16)
    rs2 = _recip(s2)
    heads_of2 = [[2 * c + g // 4 for g in range(8)] for c in range(4)]
    p2 = _pass_b(src, dst, ehat2, h2tabs, z128, heads_of=heads_of2)

    parts = []
    for c in range(4):
        parts.append(p2[c][0])
        parts.append(p2[c][1])
    return _final(parts, rs2, exp2, m128, b2mean)
